# Initial kernel scaffold; baseline (speedup 1.0000x reference)
import jax
import jax.numpy as jnp
from jax.experimental import pallas as pl


def kernel(inputs, edge_index, W, b):
    raise NotImplementedError("write your pallas kernel here")


# speedup vs baseline: 5.0839x; 5.0839x over previous
import functools

import jax
import jax.numpy as jnp
from jax import lax
from jax.experimental import pallas as pl
from jax.experimental.pallas import tpu as pltpu
from jax.experimental.pallas import tpu_sc as plsc

N_NODES = 10000
N_EDGES = 320000
D = 128
DA = D + 16

NC = 2
NS = 16
NW = NC * NS
CHUNK = 128
CHUNKS_PER_W = N_EDGES // (NW * CHUNK) + 1
E_PAD = NW * CHUNK * CHUNKS_PER_W
N_PAD = 10112
ROWS_PER_TILE = N_PAD // NS
TRASH_ROW = N_NODES + 8


def _sc_aggregate(inputs_aug, src3, dst3):
    mesh = plsc.VectorSubcoreMesh(core_axis_name="c", subcore_axis_name="s")

    @functools.partial(
        pl.kernel,
        out_type=jax.ShapeDtypeStruct((NC, N_PAD, DA), jnp.float32),
        mesh=mesh,
        scratch_types=[
            pltpu.VMEM((CHUNKS_PER_W, CHUNK), jnp.int32),
            pltpu.VMEM((CHUNKS_PER_W, CHUNK), jnp.int32),
            pltpu.VMEM((CHUNK, DA), jnp.float32),
            pltpu.VMEM_SHARED((N_PAD, DA), jnp.float32),
            pltpu.SemaphoreType.DMA,
        ],
        compiler_params=pltpu.CompilerParams(use_tc_tiling_on_sc=False),
    )
    def sc_kernel(inputs_hbm, src_hbm, dst_hbm, acc_out,
                  src_v, dst_v, rows_v, acc_s, sem):
        cid = lax.axis_index("c")
        sid = lax.axis_index("s")
        wid = cid * NS + sid
        row0 = sid * ROWS_PER_TILE

        z16 = jnp.zeros((16,), jnp.float32)

        def fill_rows(i, _):
            for c in range(DA // 16):
                rows_v[i, pl.ds(c * 16, 16)] = z16
            return 0
        lax.fori_loop(0, CHUNK, fill_rows, 0)

        for t in range(ROWS_PER_TILE // CHUNK):
            pltpu.sync_copy(rows_v, acc_s.at[pl.ds(row0 + t * CHUNK, CHUNK)])
        rem = ROWS_PER_TILE % CHUNK
        if rem:
            pltpu.sync_copy(rows_v.at[pl.ds(0, rem)],
                            acc_s.at[pl.ds(row0 + ROWS_PER_TILE - rem, rem)])
        plsc.subcore_barrier()

        pltpu.sync_copy(src_hbm.at[wid], src_v)
        pltpu.sync_copy(dst_hbm.at[wid], dst_v)

        def body(j, _):
            pltpu.async_copy(inputs_hbm.at[src_v.at[j]], rows_v, sem).wait()
            pltpu.sync_copy(rows_v, acc_s.at[dst_v.at[j]], add=True)
            return 0
        lax.fori_loop(0, CHUNKS_PER_W, body, 0)
        plsc.subcore_barrier()

        pltpu.sync_copy(acc_s.at[pl.ds(row0, ROWS_PER_TILE)],
                        acc_out.at[cid, pl.ds(row0, ROWS_PER_TILE)])

    return sc_kernel(inputs_aug, src3, dst3)


def _tc_finish(acc, W, b2):
    BLK = 400
    grid = N_NODES // BLK

    def tc_kernel(acc_ref, w_ref, b_ref, o_ref):
        s = acc_ref[0] + acc_ref[1]
        a = s[:, :D]
        c = jnp.max(s[:, D:], axis=1)
        h = a / jnp.maximum(c, 1.0)[:, None]
        o_ref[...] = lax.dot_general(
            h, w_ref[...], (((1,), (1,)), ((), ())),
            preferred_element_type=jnp.float32) + b_ref[...]

    return pl.pallas_call(
        tc_kernel,
        grid=(grid,),
        in_specs=[
            pl.BlockSpec((NC, BLK, DA), lambda i: (0, i, 0)),
            pl.BlockSpec((D, D), lambda i: (0, 0)),
            pl.BlockSpec((1, D), lambda i: (0, 0)),
        ],
        out_specs=pl.BlockSpec((BLK, D), lambda i: (i, 0)),
        out_shape=jax.ShapeDtypeStruct((N_NODES, D), jnp.float32),
    )(acc, W, b2)


def kernel(inputs, edge_index, W, b):
    inputs_aug = jnp.concatenate(
        [inputs, jnp.ones((N_NODES, DA - D), jnp.float32)], axis=1)
    src = edge_index[0].astype(jnp.int32)
    dst = edge_index[1].astype(jnp.int32)
    pad = E_PAD - N_EDGES
    src = jnp.concatenate([src, jnp.zeros((pad,), jnp.int32)])
    dst = jnp.concatenate([dst, jnp.full((pad,), TRASH_ROW, jnp.int32)])
    src3 = src.reshape(NW, CHUNKS_PER_W, CHUNK)
    dst3 = dst.reshape(NW, CHUNKS_PER_W, CHUNK)
    acc = _sc_aggregate(inputs_aug, src3, dst3)
    return _tc_finish(acc, W, b.reshape(1, D))
